# Initial kernel scaffold; baseline (speedup 1.0000x reference)
import jax
import jax.numpy as jnp
from jax.experimental import pallas as pl


def kernel(reals, cardIDs, card_nums, actionIDs, action_mask, embed_table, avg_reals, var_reals, avg_cards, var_cards):
    raise NotImplementedError("write your pallas kernel here")


# speedup vs baseline: 1.9164x; 1.9164x over previous
import jax
import jax.numpy as jnp
from jax import lax
from jax.experimental import pallas as pl
from jax.experimental.pallas import tpu as pltpu
from jax.experimental.pallas import tpu_sc as plsc

_B = 4096
_D = 32
_NCARD = 200
_NCR = 16
_NACT = 50
_ADEPTH = 4
_R = _B * _NCARD

_NW = 32
_PW = _R // _NW
_CH = 1024
_NSUB = _CH // 128
_NCHUNK = _PW // _CH


def _sc_body(cards_ref, acts_ref, nums_ref, table_ref, sb_ref,
             card_out_ref, act_out_ref,
             idx_v, rows_v, nums_v, sb_v, gsem):
  wid = lax.axis_index("s") * 2 + lax.axis_index("c")
  base = wid * _PW
  base2d = wid * (_PW // 128)

  pltpu.sync_copy(sb_ref, sb_v)
  scale = sb_v[0]
  bias = sb_v[1]

  def card_chunk(i, carry):
    row = base + i * _CH
    row2d = base2d + i * _NSUB
    pltpu.sync_copy(cards_ref.at[pl.ds(row2d, _NSUB)], idx_v)
    cps = [pltpu.async_copy(table_ref.at[idx_v.at[j]],
                            rows_v.at[pl.ds(j * 128, 128)], gsem)
           for j in range(_NSUB)]
    pltpu.sync_copy(nums_ref.at[pl.ds(row, _CH)], nums_v)

    def norm(k, c):
      nums_v[k] = nums_v[k] * scale + bias
      return c
    lax.fori_loop(0, _CH, norm, 0, unroll=8)

    for cp in cps:
      cp.wait()
    pltpu.sync_copy(rows_v, card_out_ref.at[pl.ds(row, _CH), pl.ds(0, _D)])
    pltpu.sync_copy(nums_v, card_out_ref.at[pl.ds(row, _CH), pl.ds(_D, _NCR)])
    return carry

  lax.fori_loop(0, _NCHUNK, card_chunk, 0)

  def act_chunk(i, carry):
    row = base + i * _CH
    row2d = base2d + i * _NSUB
    pltpu.sync_copy(acts_ref.at[pl.ds(row2d, _NSUB)], idx_v)
    cps = [pltpu.async_copy(table_ref.at[idx_v.at[j]],
                            rows_v.at[pl.ds(j * 128, 128)], gsem)
           for j in range(_NSUB)]
    for cp in cps:
      cp.wait()
    pltpu.sync_copy(rows_v, act_out_ref.at[pl.ds(row, _CH)])
    return carry

  lax.fori_loop(0, _NCHUNK, act_chunk, 0)


def _sc_call(cards2, acts2, nums2, table, sb):
  mesh = plsc.VectorSubcoreMesh(core_axis_name="c", subcore_axis_name="s",
                                num_cores=2, num_subcores=16)
  f = pl.kernel(
      _sc_body,
      out_type=(jax.ShapeDtypeStruct((_R, _D + _NCR), jnp.float32),
                jax.ShapeDtypeStruct((_R, _D), jnp.float32)),
      mesh=mesh,
      compiler_params=pltpu.CompilerParams(use_tc_tiling_on_sc=False),
      scratch_types=(
          pltpu.VMEM((_NSUB, 128), jnp.int32),
          pltpu.VMEM((_CH, _D), jnp.float32),
          pltpu.VMEM((_CH, _NCR), jnp.float32),
          pltpu.VMEM((2, _NCR), jnp.float32),
          pltpu.SemaphoreType.DMA,
      ),
  )
  return f(cards2, acts2, nums2, table, sb)


def _reals_body(r_ref, a_ref, v_ref, o_ref):
  o_ref[...] = (r_ref[...] - a_ref[...]) / jnp.sqrt(v_ref[...])


def _reals_norm(reals, avg, var):
  return pl.pallas_call(
      _reals_body,
      out_shape=jax.ShapeDtypeStruct(reals.shape, reals.dtype),
  )(reals, avg, var)


def kernel(reals, cardIDs, card_nums, actionIDs, action_mask,
           embed_table, avg_reals, var_reals, avg_cards, var_cards):
  cards2 = cardIDs.astype(jnp.int32).reshape(_R // 128, 128)
  acts2 = actionIDs.astype(jnp.int32).reshape(_R // 128, 128)
  nums2 = card_nums.reshape(_R, _NCR)
  scale = (1.0 / jnp.sqrt(var_cards)).reshape(1, _NCR)
  bias = (-avg_cards).reshape(1, _NCR) * scale
  sb = jnp.concatenate([scale, bias], axis=0)
  card_out, act_out = _sc_call(cards2, acts2, nums2, embed_table, sb)
  reals_n = _reals_norm(reals, avg_reals, var_reals)
  card_all = card_out.reshape(_B, _NCARD, _D + _NCR)
  action_embed = act_out.reshape(_B, _NACT, _ADEPTH * _D)
  return (reals_n, card_all, action_embed, action_mask)


# speedup vs baseline: 2.0166x; 1.0523x over previous
import jax
import jax.numpy as jnp
from jax import lax
from jax.experimental import pallas as pl
from jax.experimental.pallas import tpu as pltpu
from jax.experimental.pallas import tpu_sc as plsc

_B = 4096
_D = 32
_NCARD = 200
_NCR = 16
_NACT = 50
_ADEPTH = 4
_R = _B * _NCARD

_NW = 32
_PW = _R // _NW
_CH = 512
_NSUB = _CH // 128
_NCHUNK = _PW // _CH


def _sc_body(cards_ref, acts_ref, nums_ref, table_ref, sb_ref,
             card_out_ref, act_out_ref,
             idx_all, rows0, rows1, nums0, nums1, sb_v,
             gsem0, gsem1, nsem0, nsem1, osem0, osem1):
  wid = lax.axis_index("s") * 2 + lax.axis_index("c")
  base = wid * _PW
  base2d = wid * (_PW // 128)

  pltpu.sync_copy(sb_ref, sb_v)
  scale = sb_v[0]
  bias = sb_v[1]

  def fire_gathers(c, rows_v, gsem):
    for j in range(_NSUB):
      pltpu.async_copy(table_ref.at[idx_all.at[c * _NSUB + j]],
                       rows_v.at[pl.ds(j * 128, 128)], gsem)

  def drain_gathers(rows_v, gsem):
    pltpu.make_async_copy(table_ref.at[pl.ds(0, _CH)], rows_v, gsem).wait()

  def fire_nums(c, nums_v, nsem):
    pltpu.async_copy(nums_ref.at[pl.ds(base + c * _CH, _CH)], nums_v, nsem)

  def drain_nums(nums_v, nsem):
    pltpu.make_async_copy(nums_ref.at[pl.ds(0, _CH)], nums_v, nsem).wait()

  def norm(nums_v):
    def body(k, carry):
      nums_v[k] = nums_v[k] * scale + bias
      return carry
    lax.fori_loop(0, _CH, body, 0, unroll=8)

  pltpu.sync_copy(cards_ref.at[pl.ds(base2d, _PW // 128)], idx_all)

  def card_start(c, rows_v, nums_v, gsem, nsem):
    fire_gathers(c, rows_v, gsem)
    fire_nums(c, nums_v, nsem)

  def card_finish(c, rows_v, nums_v, gsem, nsem, osem):
    row = base + c * _CH
    drain_nums(nums_v, nsem)
    norm(nums_v)
    drain_gathers(rows_v, gsem)
    o1 = pltpu.async_copy(rows_v, card_out_ref.at[pl.ds(row, _CH), pl.ds(0, _D)], osem)
    o2 = pltpu.async_copy(nums_v, card_out_ref.at[pl.ds(row, _CH), pl.ds(_D, _NCR)], osem)
    return (o1, o2)

  card_start(0, rows0, nums0, gsem0, nsem0)
  card_start(1, rows1, nums1, gsem1, nsem1)

  def card_pair(i, carry):
    c = 2 * i
    outs = card_finish(c, rows0, nums0, gsem0, nsem0, osem0)
    for o in outs:
      o.wait()
    card_start(c + 2, rows0, nums0, gsem0, nsem0)
    outs = card_finish(c + 1, rows1, nums1, gsem1, nsem1, osem1)
    for o in outs:
      o.wait()
    card_start(c + 3, rows1, nums1, gsem1, nsem1)
    return carry

  lax.fori_loop(0, _NCHUNK // 2 - 1, card_pair, 0)

  for o in card_finish(_NCHUNK - 2, rows0, nums0, gsem0, nsem0, osem0):
    o.wait()
  for o in card_finish(_NCHUNK - 1, rows1, nums1, gsem1, nsem1, osem1):
    o.wait()

  pltpu.sync_copy(acts_ref.at[pl.ds(base2d, _PW // 128)], idx_all)

  def act_finish(c, rows_v, gsem, osem):
    row = base + c * _CH
    drain_gathers(rows_v, gsem)
    return pltpu.async_copy(rows_v, act_out_ref.at[pl.ds(row, _CH)], osem)

  fire_gathers(0, rows0, gsem0)
  fire_gathers(1, rows1, gsem1)

  def act_pair(i, carry):
    c = 2 * i
    act_finish(c, rows0, gsem0, osem0).wait()
    fire_gathers(c + 2, rows0, gsem0)
    act_finish(c + 1, rows1, gsem1, osem1).wait()
    fire_gathers(c + 3, rows1, gsem1)
    return carry

  lax.fori_loop(0, _NCHUNK // 2 - 1, act_pair, 0)

  act_finish(_NCHUNK - 2, rows0, gsem0, osem0).wait()
  act_finish(_NCHUNK - 1, rows1, gsem1, osem1).wait()


def _sc_call(cards2, acts2, nums2, table, sb):
  mesh = plsc.VectorSubcoreMesh(core_axis_name="c", subcore_axis_name="s",
                                num_cores=2, num_subcores=16)
  f = pl.kernel(
      _sc_body,
      out_type=(jax.ShapeDtypeStruct((_R, _D + _NCR), jnp.float32),
                jax.ShapeDtypeStruct((_R, _D), jnp.float32)),
      mesh=mesh,
      compiler_params=pltpu.CompilerParams(use_tc_tiling_on_sc=False),
      scratch_types=(
          pltpu.VMEM((_PW // 128, 128), jnp.int32),
          pltpu.VMEM((_CH, _D), jnp.float32),
          pltpu.VMEM((_CH, _D), jnp.float32),
          pltpu.VMEM((_CH, _NCR), jnp.float32),
          pltpu.VMEM((_CH, _NCR), jnp.float32),
          pltpu.VMEM((2, _NCR), jnp.float32),
          pltpu.SemaphoreType.DMA,
          pltpu.SemaphoreType.DMA,
          pltpu.SemaphoreType.DMA,
          pltpu.SemaphoreType.DMA,
          pltpu.SemaphoreType.DMA,
          pltpu.SemaphoreType.DMA,
      ),
  )
  return f(cards2, acts2, nums2, table, sb)


def _reals_body(r_ref, a_ref, v_ref, o_ref):
  o_ref[...] = (r_ref[...] - a_ref[...]) / jnp.sqrt(v_ref[...])


def _reals_norm(reals, avg, var):
  return pl.pallas_call(
      _reals_body,
      out_shape=jax.ShapeDtypeStruct(reals.shape, reals.dtype),
  )(reals, avg, var)


def kernel(reals, cardIDs, card_nums, actionIDs, action_mask,
           embed_table, avg_reals, var_reals, avg_cards, var_cards):
  cards2 = cardIDs.astype(jnp.int32).reshape(_R // 128, 128)
  acts2 = actionIDs.astype(jnp.int32).reshape(_R // 128, 128)
  nums2 = card_nums.reshape(_R, _NCR)
  scale = (1.0 / jnp.sqrt(var_cards)).reshape(1, _NCR)
  bias = (-avg_cards).reshape(1, _NCR) * scale
  sb = jnp.concatenate([scale, bias], axis=0)
  card_out, act_out = _sc_call(cards2, acts2, nums2, embed_table, sb)
  reals_n = _reals_norm(reals, avg_reals, var_reals)
  card_all = card_out.reshape(_B, _NCARD, _D + _NCR)
  action_embed = act_out.reshape(_B, _NACT, _ADEPTH * _D)
  return (reals_n, card_all, action_embed, action_mask)


# speedup vs baseline: 2.0621x; 1.0226x over previous
import jax
import jax.numpy as jnp
from jax import lax
from jax.experimental import pallas as pl
from jax.experimental.pallas import tpu as pltpu
from jax.experimental.pallas import tpu_sc as plsc

_B = 4096
_D = 32
_NCARD = 200
_NCR = 16
_NACT = 50
_ADEPTH = 4
_R = _B * _NCARD

_NW = 32
_CH = 512
_NSUB = _CH // 128
_BC = _B // _CH
_UNITS = _R // _CH // _NW
_IROWS = _UNITS * _NSUB


def _sc_body(cards_ref, acts_ref, nums_ref, table_ref, sb_ref,
             card_out_ref, act_out_ref,
             idx_all, g0, g1, s0, s1, sb_v,
             gsem0, gsem1, nsem0, nsem1, osem0, osem1):
  wid = lax.axis_index("s") * 2 + lax.axis_index("c")
  ubase = wid * _UNITS
  irow0 = wid * _IROWS

  pltpu.sync_copy(sb_ref, sb_v)
  iota = lax.iota(jnp.int32, 16)

  def fire_gathers(u, rows_v, gsem):
    for j in range(_NSUB):
      pltpu.async_copy(table_ref.at[idx_all.at[u * _NSUB + j]],
                       rows_v.at[pl.ds(j * 128, 128)], gsem)

  def drain_gathers(rows_v, gsem):
    pltpu.make_async_copy(table_ref.at[pl.ds(0, _CH)], rows_v, gsem).wait()

  pltpu.sync_copy(cards_ref.at[pl.ds(irow0, _IROWS)], idx_all)

  def card_start(u, rows_v, s_v, gsem, nsem):
    g = ubase + u
    c = g >> 3
    b0 = (g & 7) * _CH
    fire_gathers(u, rows_v, gsem)
    pltpu.async_copy(nums_ref.at[c, :, pl.ds(b0, _CH)],
                     s_v.at[pl.ds(_D, _NCR)], nsem)

  def card_finish(u, rows_v, s_v, gsem, nsem, osem):
    g = ubase + u
    c = g >> 3
    b0 = (g & 7) * _CH
    pltpu.make_async_copy(nums_ref.at[0, :, pl.ds(0, _CH)],
                          s_v.at[pl.ds(_D, _NCR)], nsem).wait()
    drain_gathers(rows_v, gsem)

    def unit_body(cb, carry):
      ridx = iota + cb * 16
      for f in range(_D):
        v = plsc.load_gather(rows_v, [ridx, jnp.full((16,), f, jnp.int32)])
        s_v[f, pl.ds(cb * 16, 16)] = v
      for f in range(_NCR):
        x = s_v[_D + f, pl.ds(cb * 16, 16)]
        s_v[_D + f, pl.ds(cb * 16, 16)] = x * sb_v[0, f] + sb_v[1, f]
      return carry
    lax.fori_loop(0, _CH // 16, unit_body, 0)

    return pltpu.async_copy(s_v, card_out_ref.at[c, :, pl.ds(b0, _CH)], osem)

  card_start(0, g0, s0, gsem0, nsem0)
  card_start(1, g1, s1, gsem1, nsem1)

  def card_pair(i, carry):
    u = 2 * i
    card_finish(u, g0, s0, gsem0, nsem0, osem0).wait()
    card_start(u + 2, g0, s0, gsem0, nsem0)
    card_finish(u + 1, g1, s1, gsem1, nsem1, osem1).wait()
    card_start(u + 3, g1, s1, gsem1, nsem1)
    return carry

  lax.fori_loop(0, _UNITS // 2 - 1, card_pair, 0)
  card_finish(_UNITS - 2, g0, s0, gsem0, nsem0, osem0).wait()
  card_finish(_UNITS - 1, g1, s1, gsem1, nsem1, osem1).wait()

  pltpu.sync_copy(acts_ref.at[pl.ds(irow0, _IROWS)], idx_all)

  def act_out(u, rows_v, osem):
    g = ubase + u
    a = g >> 5
    d = (g >> 3) & 3
    b0 = (g & 7) * _CH
    return pltpu.async_copy(
        rows_v, act_out_ref.at[a, pl.ds(b0, _CH), pl.ds(d * _D, _D)], osem)

  fire_gathers(0, g0, gsem0)
  fire_gathers(1, g1, gsem1)

  def act_pair(i, carry):
    u = 2 * i
    drain_gathers(g0, gsem0)
    act_out(u, g0, osem0).wait()
    fire_gathers(u + 2, g0, gsem0)
    drain_gathers(g1, gsem1)
    act_out(u + 1, g1, osem1).wait()
    fire_gathers(u + 3, g1, gsem1)
    return carry

  lax.fori_loop(0, _UNITS // 2 - 1, act_pair, 0)
  drain_gathers(g0, gsem0)
  act_out(_UNITS - 2, g0, osem0).wait()
  drain_gathers(g1, gsem1)
  act_out(_UNITS - 1, g1, osem1).wait()


def _sc_call(cards2, acts2, nums_t, table, sb):
  mesh = plsc.VectorSubcoreMesh(core_axis_name="c", subcore_axis_name="s",
                                num_cores=2, num_subcores=16)
  f = pl.kernel(
      _sc_body,
      out_type=(jax.ShapeDtypeStruct((_NCARD, _D + _NCR, _B), jnp.float32),
                jax.ShapeDtypeStruct((_NACT, _B, _ADEPTH * _D), jnp.float32)),
      mesh=mesh,
      compiler_params=pltpu.CompilerParams(use_tc_tiling_on_sc=False,
                                           needs_layout_passes=False),
      scratch_types=(
          pltpu.VMEM((_IROWS, 128), jnp.int32),
          pltpu.VMEM((_CH, _D), jnp.float32),
          pltpu.VMEM((_CH, _D), jnp.float32),
          pltpu.VMEM((_D + _NCR, _CH), jnp.float32),
          pltpu.VMEM((_D + _NCR, _CH), jnp.float32),
          pltpu.VMEM((2, _NCR, 16), jnp.float32),
          pltpu.SemaphoreType.DMA,
          pltpu.SemaphoreType.DMA,
          pltpu.SemaphoreType.DMA,
          pltpu.SemaphoreType.DMA,
          pltpu.SemaphoreType.DMA,
          pltpu.SemaphoreType.DMA,
      ),
  )
  return f(cards2, acts2, nums_t, table, sb)


def _reals_body(r_ref, a_ref, v_ref, o_ref):
  o_ref[...] = (r_ref[...] - a_ref[...]) / jnp.sqrt(v_ref[...])


def _reals_norm(reals, avg, var):
  return pl.pallas_call(
      _reals_body,
      out_shape=jax.ShapeDtypeStruct(reals.shape, reals.dtype),
  )(reals, avg, var)


def kernel(reals, cardIDs, card_nums, actionIDs, action_mask,
           embed_table, avg_reals, var_reals, avg_cards, var_cards):
  cards2 = cardIDs.astype(jnp.int32).T.reshape(_R // 128, 128)
  acts2 = actionIDs.astype(jnp.int32).transpose(1, 2, 0).reshape(_R // 128, 128)
  nums_t = card_nums.transpose(1, 2, 0)
  scale = (1.0 / jnp.sqrt(var_cards)).reshape(_NCR, 1)
  bias = (-avg_cards).reshape(_NCR, 1) * scale
  sb = jnp.stack([jnp.tile(scale, (1, 16)), jnp.tile(bias, (1, 16))])
  card_out, act_out = _sc_call(cards2, acts2, nums_t, embed_table, sb)
  reals_n = _reals_norm(reals, avg_reals, var_reals)
  card_all = card_out.transpose(2, 0, 1)
  action_embed = act_out.transpose(1, 0, 2)
  return (reals_n, card_all, action_embed, action_mask)


# speedup vs baseline: 2.7348x; 1.3262x over previous
import jax
import jax.numpy as jnp
from jax import lax
from jax.experimental import pallas as pl
from jax.experimental.pallas import tpu as pltpu
from jax.experimental.pallas import tpu_sc as plsc

_B = 4096
_D = 32
_NCARD = 200
_NCR = 16
_NACT = 50
_ADEPTH = 4
_R = _B * _NCARD

_NW = 32
_CH = 512
_NSUB = _CH // 128
_UNITS = _R // _CH // _NW
_IROWS = _UNITS * _NSUB


def _sc_body(cards_ref, acts_ref, table_ref,
             card_out_ref, act_out_ref,
             idx_all, g0, g1,
             gsem0, gsem1, osem0, osem1):
  wid = lax.axis_index("s") * 2 + lax.axis_index("c")
  ubase = wid * _UNITS
  irow0 = wid * _IROWS

  def fire_gathers(u, rows_v, gsem):
    for j in range(_NSUB):
      pltpu.async_copy(table_ref.at[idx_all.at[u * _NSUB + j]],
                       rows_v.at[pl.ds(j * 128, 128)], gsem)

  def drain_gathers(rows_v, gsem):
    pltpu.make_async_copy(table_ref.at[pl.ds(0, _CH)], rows_v, gsem).wait()

  def card_out(u, rows_v, osem):
    g = ubase + u
    c = g >> 3
    b0 = (g & 7) * _CH
    return pltpu.async_copy(rows_v, card_out_ref.at[c, pl.ds(b0, _CH)], osem)

  def act_out(u, rows_v, osem):
    g = ubase + u
    a = g >> 5
    d = (g >> 3) & 3
    b0 = (g & 7) * _CH
    return pltpu.async_copy(
        rows_v, act_out_ref.at[a, pl.ds(b0, _CH), pl.ds(d * _D, _D)], osem)

  def phase(idx_hbm, out_fn):
    pltpu.sync_copy(idx_hbm.at[pl.ds(irow0, _IROWS)], idx_all)
    fire_gathers(0, g0, gsem0)
    fire_gathers(1, g1, gsem1)

    def pair(i, carry):
      u = 2 * i
      drain_gathers(g0, gsem0)
      out_fn(u, g0, osem0).wait()
      fire_gathers(u + 2, g0, gsem0)
      drain_gathers(g1, gsem1)
      out_fn(u + 1, g1, osem1).wait()
      fire_gathers(u + 3, g1, gsem1)
      return carry

    lax.fori_loop(0, _UNITS // 2 - 1, pair, 0)
    drain_gathers(g0, gsem0)
    out_fn(_UNITS - 2, g0, osem0).wait()
    drain_gathers(g1, gsem1)
    out_fn(_UNITS - 1, g1, osem1).wait()

  phase(cards_ref, card_out)
  phase(acts_ref, act_out)


def _sc_call(cards2, acts2, table):
  mesh = plsc.VectorSubcoreMesh(core_axis_name="c", subcore_axis_name="s",
                                num_cores=2, num_subcores=16)
  f = pl.kernel(
      _sc_body,
      out_type=(jax.ShapeDtypeStruct((_NCARD, _B, _D), jnp.float32),
                jax.ShapeDtypeStruct((_NACT, _B, _ADEPTH * _D), jnp.float32)),
      mesh=mesh,
      compiler_params=pltpu.CompilerParams(use_tc_tiling_on_sc=False,
                                           needs_layout_passes=False),
      scratch_types=(
          pltpu.VMEM((_IROWS, 128), jnp.int32),
          pltpu.VMEM((_CH, _D), jnp.float32),
          pltpu.VMEM((_CH, _D), jnp.float32),
          pltpu.SemaphoreType.DMA,
          pltpu.SemaphoreType.DMA,
          pltpu.SemaphoreType.DMA,
          pltpu.SemaphoreType.DMA,
      ),
  )
  return f(cards2, acts2, table)


def _card_tc_body(e_ref, n_ref, s_ref, b_ref, o_ref):
  emb = e_ref[0]
  o_ref[0, pl.ds(0, _D), :] = emb.T
  o_ref[0, pl.ds(_D, _NCR), :] = n_ref[0] * s_ref[...] + b_ref[...]


def _card_tc(card_embed_t, nums_t, scale, bias):
  return pl.pallas_call(
      _card_tc_body,
      grid=(_NCARD,),
      in_specs=[
          pl.BlockSpec((1, _B, _D), lambda c: (c, 0, 0)),
          pl.BlockSpec((1, _NCR, _B), lambda c: (c, 0, 0)),
          pl.BlockSpec((_NCR, 1), lambda c: (0, 0)),
          pl.BlockSpec((_NCR, 1), lambda c: (0, 0)),
      ],
      out_specs=pl.BlockSpec((1, _D + _NCR, _B), lambda c: (c, 0, 0)),
      out_shape=jax.ShapeDtypeStruct((_NCARD, _D + _NCR, _B), jnp.float32),
  )(card_embed_t, nums_t, scale, bias)


def _reals_body(r_ref, a_ref, v_ref, o_ref):
  o_ref[...] = (r_ref[...] - a_ref[...]) / jnp.sqrt(v_ref[...])


def _reals_norm(reals, avg, var):
  return pl.pallas_call(
      _reals_body,
      out_shape=jax.ShapeDtypeStruct(reals.shape, reals.dtype),
  )(reals, avg, var)


def kernel(reals, cardIDs, card_nums, actionIDs, action_mask,
           embed_table, avg_reals, var_reals, avg_cards, var_cards):
  cards2 = cardIDs.astype(jnp.int32).T.reshape(_R // 128, 128)
  acts2 = actionIDs.astype(jnp.int32).transpose(1, 2, 0).reshape(_R // 128, 128)
  nums_t = card_nums.transpose(1, 2, 0)
  scale = (1.0 / jnp.sqrt(var_cards)).reshape(_NCR, 1)
  bias = (-avg_cards).reshape(_NCR, 1) * scale
  card_embed_t, act_out = _sc_call(cards2, acts2, embed_table)
  card_out = _card_tc(card_embed_t, nums_t, scale, bias)
  reals_n = _reals_norm(reals, avg_reals, var_reals)
  card_all = card_out.transpose(2, 0, 1)
  action_embed = act_out.transpose(1, 0, 2)
  return (reals_n, card_all, action_embed, action_mask)


# speedup vs baseline: 2.7718x; 1.0135x over previous
import jax
import jax.numpy as jnp
from jax import lax
from jax.experimental import pallas as pl
from jax.experimental.pallas import tpu as pltpu
from jax.experimental.pallas import tpu_sc as plsc

_B = 4096
_D = 32
_NCARD = 200
_NCR = 16
_NACT = 50
_ADEPTH = 4
_R = _B * _NCARD

_NW = 32
_CH = 512
_NSUB = _CH // 128
_UNITS = _R // _CH // _NW
_IROWS = _UNITS * _NSUB


def _sc_body(cards_ref, acts_ref, table_ref,
             card_out_ref, act_out_ref,
             idx_all, g0, g1,
             gsem0, gsem1, osem0, osem1):
  wid = lax.axis_index("s") * 2 + lax.axis_index("c")
  ubase = wid * _UNITS
  irow0 = wid * _IROWS

  def fire_gathers(u, rows_v, gsem):
    for j in range(_NSUB):
      pltpu.async_copy(table_ref.at[idx_all.at[u * _NSUB + j]],
                       rows_v.at[pl.ds(j * 128, 128)], gsem)

  def drain_gathers(rows_v, gsem):
    pltpu.make_async_copy(table_ref.at[pl.ds(0, _CH)], rows_v, gsem).wait()

  def card_out(u, rows_v, osem):
    g = ubase + u
    c = g >> 3
    b0 = (g & 7) * _CH
    return pltpu.async_copy(rows_v, card_out_ref.at[c, pl.ds(b0, _CH)], osem)

  def act_out(u, rows_v, osem):
    g = ubase + u
    a = g >> 5
    d = (g >> 3) & 3
    b0 = (g & 7) * _CH
    return pltpu.async_copy(
        rows_v, act_out_ref.at[a, pl.ds(b0, _CH), pl.ds(d * _D, _D)], osem)

  def phase(idx_hbm, out_fn):
    pltpu.sync_copy(idx_hbm.at[pl.ds(irow0, _IROWS)], idx_all)
    fire_gathers(0, g0, gsem0)
    fire_gathers(1, g1, gsem1)

    def pair(i, carry):
      u = 2 * i
      drain_gathers(g0, gsem0)
      out_fn(u, g0, osem0).wait()
      fire_gathers(u + 2, g0, gsem0)
      drain_gathers(g1, gsem1)
      out_fn(u + 1, g1, osem1).wait()
      fire_gathers(u + 3, g1, gsem1)
      return carry

    lax.fori_loop(0, _UNITS // 2 - 1, pair, 0)
    drain_gathers(g0, gsem0)
    out_fn(_UNITS - 2, g0, osem0).wait()
    drain_gathers(g1, gsem1)
    out_fn(_UNITS - 1, g1, osem1).wait()

  phase(cards_ref, card_out)
  phase(acts_ref, act_out)


def _sc_call(cards2, acts2, table):
  mesh = plsc.VectorSubcoreMesh(core_axis_name="c", subcore_axis_name="s",
                                num_cores=2, num_subcores=16)
  f = pl.kernel(
      _sc_body,
      out_type=(jax.ShapeDtypeStruct((_NCARD, _B, _D), jnp.float32),
                jax.ShapeDtypeStruct((_NACT, _B, _ADEPTH * _D), jnp.float32)),
      mesh=mesh,
      compiler_params=pltpu.CompilerParams(use_tc_tiling_on_sc=False,
                                           needs_layout_passes=False),
      scratch_types=(
          pltpu.VMEM((_IROWS, 128), jnp.int32),
          pltpu.VMEM((_CH, _D), jnp.float32),
          pltpu.VMEM((_CH, _D), jnp.float32),
          pltpu.SemaphoreType.DMA,
          pltpu.SemaphoreType.DMA,
          pltpu.SemaphoreType.DMA,
          pltpu.SemaphoreType.DMA,
      ),
  )
  return f(cards2, acts2, table)


def _card_tc_body(e_ref, n_ref, s_ref, b_ref, o_ref):
  emb = e_ref[0]
  ident = jnp.eye(_D, dtype=jnp.float32)
  o_ref[0, pl.ds(0, _D), :] = lax.dot_general(
      ident, emb, (((1,), (1,)), ((), ())),
      preferred_element_type=jnp.float32)
  o_ref[0, pl.ds(_D, _NCR), :] = n_ref[0] * s_ref[...] + b_ref[...]


def _card_tc(card_embed_t, nums_t, scale, bias):
  return pl.pallas_call(
      _card_tc_body,
      grid=(_NCARD,),
      in_specs=[
          pl.BlockSpec((1, _B, _D), lambda c: (c, 0, 0)),
          pl.BlockSpec((1, _NCR, _B), lambda c: (c, 0, 0)),
          pl.BlockSpec((_NCR, 1), lambda c: (0, 0)),
          pl.BlockSpec((_NCR, 1), lambda c: (0, 0)),
      ],
      out_specs=pl.BlockSpec((1, _D + _NCR, _B), lambda c: (c, 0, 0)),
      out_shape=jax.ShapeDtypeStruct((_NCARD, _D + _NCR, _B), jnp.float32),
  )(card_embed_t, nums_t, scale, bias)


def _reals_body(r_ref, a_ref, v_ref, o_ref):
  o_ref[...] = (r_ref[...] - a_ref[...]) / jnp.sqrt(v_ref[...])


def _reals_norm(reals, avg, var):
  return pl.pallas_call(
      _reals_body,
      out_shape=jax.ShapeDtypeStruct(reals.shape, reals.dtype),
  )(reals, avg, var)


def kernel(reals, cardIDs, card_nums, actionIDs, action_mask,
           embed_table, avg_reals, var_reals, avg_cards, var_cards):
  cards2 = cardIDs.astype(jnp.int32).T.reshape(_R // 128, 128)
  acts2 = actionIDs.astype(jnp.int32).transpose(1, 2, 0).reshape(_R // 128, 128)
  nums_t = card_nums.transpose(1, 2, 0)
  scale = (1.0 / jnp.sqrt(var_cards)).reshape(_NCR, 1)
  bias = (-avg_cards).reshape(_NCR, 1) * scale
  card_embed_t, act_out = _sc_call(cards2, acts2, embed_table)
  card_out = _card_tc(card_embed_t, nums_t, scale, bias)
  reals_n = _reals_norm(reals, avg_reals, var_reals)
  card_all = card_out.transpose(2, 0, 1)
  action_embed = act_out.transpose(1, 0, 2)
  return (reals_n, card_all, action_embed, action_mask)
